# Initial kernel scaffold; baseline (speedup 1.0000x reference)
import jax
import jax.numpy as jnp
from jax.experimental import pallas as pl


def kernel(h, edge_index, gamma, beta):
    raise NotImplementedError("write your pallas kernel here")


# speedup vs baseline: 5.5513x; 5.5513x over previous
import functools

import jax
import jax.numpy as jnp
from jax import lax
from jax.experimental import pallas as pl
from jax.experimental.pallas import tpu as pltpu
from jax.experimental.pallas import tpu_sc as plsc

N_NODES = 10000
N_EDGES = 320000
D = 128
DH = D // 2
EPS = 1e-5

NC = 2
NS = 16
NW = NC * NS
EPW = N_EDGES // NW
CHUNK = 80
NCHUNK = EPW // CHUNK
ZCH = 80
NODE_CHUNKS = N_NODES // ZCH


def _sc_segment_sum(h2, src3, dst3):
    mesh = plsc.VectorSubcoreMesh(core_axis_name="c", subcore_axis_name="s")

    @functools.partial(
        pl.kernel,
        out_type=jax.ShapeDtypeStruct((2, NC, N_NODES, DH), jnp.float32),
        mesh=mesh,
        compiler_params=pltpu.CompilerParams(use_tc_tiling_on_sc=False),
        scratch_types=[
            pltpu.VMEM((NCHUNK, CHUNK), jnp.int32),
            pltpu.VMEM((NCHUNK, CHUNK), jnp.int32),
            pltpu.VMEM((CHUNK, DH), jnp.float32),
            pltpu.VMEM((ZCH, DH), jnp.float32),
            pltpu.VMEM_SHARED((N_NODES, DH), jnp.float32),
            pltpu.SemaphoreType.DMA,
        ],
    )
    def k(h_hbm, src_hbm, dst_hbm, out_hbm, src_v, dst_v, rows_v, zbuf, acc, sem):
        c = lax.axis_index("c")
        s = lax.axis_index("s")
        wid = c * NS + s

        pltpu.sync_copy(src_hbm.at[wid], src_v)
        pltpu.sync_copy(dst_hbm.at[wid], dst_v)

        def zstore(i, carry):
            zbuf[i // (DH // 16), pl.ds((i % (DH // 16)) * 16, 16)] = (
                jnp.zeros((16,), jnp.float32))
            return carry
        lax.fori_loop(0, ZCH * (DH // 16), zstore, 0)

        def zero_acc():
            def zinit(i, carry):
                cid = s + i * NS

                @pl.when(cid < NODE_CHUNKS)
                def _():
                    pltpu.sync_copy(zbuf, acc.at[pl.ds(cid * ZCH, ZCH)])
                return carry
            lax.fori_loop(0, (NODE_CHUNKS + NS - 1) // NS, zinit, 0)

        for p in range(2):
            zero_acc()
            plsc.subcore_barrier()

            def chunk(j, carry):
                pltpu.async_copy(h_hbm.at[src_v.at[j]], rows_v, sem).wait()
                pltpu.sync_copy(rows_v, acc.at[dst_v.at[j]], add=True)
                return carry
            lax.fori_loop(0, NCHUNK, chunk, 0)
            plsc.subcore_barrier()

            def wout(i, carry):
                cid = s + i * NS

                @pl.when(cid < NODE_CHUNKS)
                def _():
                    pltpu.sync_copy(acc.at[pl.ds(cid * ZCH, ZCH)],
                                    out_hbm.at[p, c, pl.ds(cid * ZCH, ZCH)])
                return carry
            lax.fori_loop(0, (NODE_CHUNKS + NS - 1) // NS, wout, 0)

            if p == 0:
                def bump(i, carry):
                    r, q = i // (CHUNK // 16), i % (CHUNK // 16)
                    src_v[r, pl.ds(q * 16, 16)] = (
                        src_v[r, pl.ds(q * 16, 16)] + 1)
                    return carry
                lax.fori_loop(0, NCHUNK * (CHUNK // 16), bump, 0)
                plsc.subcore_barrier()

    return k(h2, src3, dst3)


def _bn_body(parts_ref, gamma_ref, beta_ref, out_ref):
    agg = jnp.concatenate(
        [parts_ref[0, 0] + parts_ref[0, 1],
         parts_ref[1, 0] + parts_ref[1, 1]], axis=1)
    mean = jnp.mean(agg, axis=0, keepdims=True)
    cent = agg - mean
    var = jnp.mean(cent * cent, axis=0, keepdims=True)
    out_ref[...] = cent * lax.rsqrt(var + EPS) * gamma_ref[...] + beta_ref[...]


def kernel(h, edge_index, gamma, beta):
    h2 = h.reshape(2 * N_NODES, DH)
    src3 = (edge_index[0] * 2).reshape(NW, NCHUNK, CHUNK)
    dst3 = edge_index[1].reshape(NW, NCHUNK, CHUNK)
    parts = _sc_segment_sum(h2, src3, dst3)
    return pl.pallas_call(
        _bn_body,
        out_shape=jax.ShapeDtypeStruct((N_NODES, D), jnp.float32),
    )(parts, gamma.reshape(1, D), beta.reshape(1, D))


# speedup vs baseline: 10.4026x; 1.8739x over previous
import functools

import jax
import jax.numpy as jnp
from jax import lax
from jax.experimental import pallas as pl
from jax.experimental.pallas import tpu as pltpu
from jax.experimental.pallas import tpu_sc as plsc

N_NODES = 10000
N_EDGES = 320000
D = 128
DH = D // 2
EPS = 1e-5

NC = 2
NS = 16
NW = NC * NS
EPW = N_EDGES // NW
CHUNK = 125
NCHUNK = EPW // CHUNK
RING = 4
ZCH = 80
NODE_CHUNKS = N_NODES // ZCH


def _sc_segment_sum(h2, srcA, srcB, dst3):
    mesh = plsc.VectorSubcoreMesh(core_axis_name="c", subcore_axis_name="s")

    @functools.partial(
        pl.kernel,
        out_type=jax.ShapeDtypeStruct((2, NC, N_NODES, DH), jnp.float32),
        mesh=mesh,
        compiler_params=pltpu.CompilerParams(use_tc_tiling_on_sc=False),
        scratch_types=[
            pltpu.VMEM((NCHUNK, CHUNK), jnp.int32),
            pltpu.VMEM((NCHUNK, CHUNK), jnp.int32),
            pltpu.VMEM((NCHUNK, CHUNK), jnp.int32),
            [pltpu.VMEM((CHUNK, DH), jnp.float32) for _ in range(RING)],
            pltpu.VMEM((ZCH, DH), jnp.float32),
            pltpu.VMEM_SHARED((N_NODES, DH), jnp.float32),
            [pltpu.SemaphoreType.DMA for _ in range(RING)],
            [pltpu.SemaphoreType.DMA for _ in range(RING)],
        ],
    )
    def k(hA_hbm, sA_hbm, sB_hbm, dst_hbm, out_hbm,
          srcA_v, srcB_v, dst_v, rows, zbuf, acc, gsem, ssem):
        c = lax.axis_index("c")
        s = lax.axis_index("s")
        wid = c * NS + s

        pltpu.sync_copy(sA_hbm.at[wid], srcA_v)
        pltpu.sync_copy(sB_hbm.at[wid], srcB_v)
        pltpu.sync_copy(dst_hbm.at[wid], dst_v)

        def zstore(i, carry):
            zbuf[i // (DH // 16), pl.ds((i % (DH // 16)) * 16, 16)] = (
                jnp.zeros((16,), jnp.float32))
            return carry
        lax.fori_loop(0, ZCH * (DH // 16), zstore, 0)

        def strided_node_chunks(body):
            def it(i, carry):
                cid = s + i * NS

                @pl.when(cid < NODE_CHUNKS)
                def _():
                    body(cid)
                return carry
            lax.fori_loop(0, (NODE_CHUNKS + NS - 1) // NS, it, 0)

        for p, src_v in ((0, srcA_v), (1, srcB_v)):
            strided_node_chunks(
                lambda cid: pltpu.sync_copy(zbuf, acc.at[pl.ds(cid * ZCH, ZCH)]))
            plsc.subcore_barrier()

            for b in range(RING):
                pltpu.async_copy(hA_hbm.at[src_v.at[b]], rows[b], gsem[b])

            def block(jb, carry):
                for b in range(RING):
                    j = jb * RING + b
                    pltpu.make_async_copy(
                        hA_hbm.at[src_v.at[j]], rows[b], gsem[b]).wait()
                    pltpu.async_copy(
                        rows[b], acc.at[dst_v.at[j]], ssem[b], add=True)
                for b in range(RING):
                    j = jb * RING + b
                    pltpu.make_async_copy(
                        rows[b], acc.at[dst_v.at[j]], ssem[b]).wait()
                    jn = j + RING

                    @pl.when(jn < NCHUNK)
                    def _():
                        pltpu.async_copy(
                            hA_hbm.at[src_v.at[jn]], rows[b], gsem[b])
                return carry
            lax.fori_loop(0, NCHUNK // RING, block, 0)
            plsc.subcore_barrier()

            strided_node_chunks(
                lambda cid: pltpu.sync_copy(
                    acc.at[pl.ds(cid * ZCH, ZCH)],
                    out_hbm.at[p, c, pl.ds(cid * ZCH, ZCH)]))
            if p == 0:
                plsc.subcore_barrier()

    return k(h2, srcA, srcB, dst3)


def _bn_body(parts_ref, gamma_ref, beta_ref, out_ref):
    agg = jnp.concatenate(
        [parts_ref[0, 0] + parts_ref[0, 1],
         parts_ref[1, 0] + parts_ref[1, 1]], axis=1)
    mean = jnp.mean(agg, axis=0, keepdims=True)
    cent = agg - mean
    var = jnp.mean(cent * cent, axis=0, keepdims=True)
    out_ref[...] = cent * lax.rsqrt(var + EPS) * gamma_ref[...] + beta_ref[...]


def kernel(h, edge_index, gamma, beta):
    h2 = h.reshape(2 * N_NODES, DH)
    srcA = (edge_index[0] * 2).reshape(NW, NCHUNK, CHUNK)
    srcB = (edge_index[0] * 2 + 1).reshape(NW, NCHUNK, CHUNK)
    dst3 = edge_index[1].reshape(NW, NCHUNK, CHUNK)
    parts = _sc_segment_sum(h2, srcA, srcB, dst3)
    return pl.pallas_call(
        _bn_body,
        out_shape=jax.ShapeDtypeStruct((N_NODES, D), jnp.float32),
    )(parts, gamma.reshape(1, D), beta.reshape(1, D))


# speedup vs baseline: 11.5220x; 1.1076x over previous
import functools

import jax
import jax.numpy as jnp
from jax import lax
from jax.experimental import pallas as pl
from jax.experimental.pallas import tpu as pltpu
from jax.experimental.pallas import tpu_sc as plsc

N_NODES = 10000
N_EDGES = 320000
D = 128
DH = D // 2
EPS = 1e-5

NC = 2
NS = 16
EPT = N_EDGES // NS
CHUNK = 125
NCHUNK = EPT // CHUNK
RING = 4
ZCH = 80
NODE_CHUNKS = N_NODES // ZCH


def _sc_segment_sum(h2, src4, dst3):
    mesh = plsc.VectorSubcoreMesh(core_axis_name="c", subcore_axis_name="s")

    @functools.partial(
        pl.kernel,
        out_type=jax.ShapeDtypeStruct((NC, N_NODES, DH), jnp.float32),
        mesh=mesh,
        compiler_params=pltpu.CompilerParams(use_tc_tiling_on_sc=False),
        scratch_types=[
            pltpu.VMEM((NCHUNK, CHUNK), jnp.int32),
            pltpu.VMEM((NCHUNK, CHUNK), jnp.int32),
            [pltpu.VMEM((CHUNK, DH), jnp.float32) for _ in range(RING)],
            pltpu.VMEM((ZCH, DH), jnp.float32),
            pltpu.VMEM_SHARED((N_NODES, DH), jnp.float32),
            [pltpu.SemaphoreType.DMA for _ in range(RING)],
            [pltpu.SemaphoreType.DMA for _ in range(RING)],
        ],
    )
    def k(h_hbm, src_hbm, dst_hbm, out_hbm,
          src_v, dst_v, rows, zbuf, acc, gsem, ssem):
        c = lax.axis_index("c")
        s = lax.axis_index("s")

        pltpu.sync_copy(src_hbm.at[c, s], src_v)
        pltpu.sync_copy(dst_hbm.at[s], dst_v)

        def zstore(i, carry):
            zbuf[i // (DH // 16), pl.ds((i % (DH // 16)) * 16, 16)] = (
                jnp.zeros((16,), jnp.float32))
            return carry
        lax.fori_loop(0, ZCH * (DH // 16), zstore, 0)

        def strided_node_chunks(body):
            def it(i, carry):
                cid = s + i * NS

                @pl.when(cid < NODE_CHUNKS)
                def _():
                    body(cid)
                return carry
            lax.fori_loop(0, (NODE_CHUNKS + NS - 1) // NS, it, 0)

        strided_node_chunks(
            lambda cid: pltpu.sync_copy(zbuf, acc.at[pl.ds(cid * ZCH, ZCH)]))
        plsc.subcore_barrier()

        for b in range(RING):
            pltpu.async_copy(h_hbm.at[src_v.at[b]], rows[b], gsem[b])

        def block(jb, carry):
            for b in range(RING):
                j = jb * RING + b
                pltpu.make_async_copy(
                    h_hbm.at[src_v.at[j]], rows[b], gsem[b]).wait()
                pltpu.async_copy(
                    rows[b], acc.at[dst_v.at[j]], ssem[b], add=True)
            for b in range(RING):
                j = jb * RING + b
                pltpu.make_async_copy(
                    rows[b], acc.at[dst_v.at[j]], ssem[b]).wait()
                jn = j + RING

                @pl.when(jn < NCHUNK)
                def _():
                    pltpu.async_copy(
                        h_hbm.at[src_v.at[jn]], rows[b], gsem[b])
            return carry
        lax.fori_loop(0, NCHUNK // RING, block, 0)
        plsc.subcore_barrier()

        strided_node_chunks(
            lambda cid: pltpu.sync_copy(
                acc.at[pl.ds(cid * ZCH, ZCH)],
                out_hbm.at[c, pl.ds(cid * ZCH, ZCH)]))

    return k(h2, src4, dst3)


def _bn_body(parts_ref, gamma_ref, beta_ref, out_ref):
    agg = jnp.concatenate([parts_ref[0], parts_ref[1]], axis=1)
    mean = jnp.mean(agg, axis=0, keepdims=True)
    cent = agg - mean
    var = jnp.mean(cent * cent, axis=0, keepdims=True)
    out_ref[...] = cent * lax.rsqrt(var + EPS) * gamma_ref[...] + beta_ref[...]


def kernel(h, edge_index, gamma, beta):
    h2 = h.reshape(2 * N_NODES, DH)
    src2 = edge_index[0] * 2
    src4 = jnp.stack([src2, src2 + 1]).reshape(NC, NS, NCHUNK, CHUNK)
    dst3 = edge_index[1].reshape(NS, NCHUNK, CHUNK)
    parts = _sc_segment_sum(h2, src4, dst3)
    return pl.pallas_call(
        _bn_body,
        out_shape=jax.ShapeDtypeStruct((N_NODES, D), jnp.float32),
    )(parts, gamma.reshape(1, D), beta.reshape(1, D))


# speedup vs baseline: 11.6391x; 1.0102x over previous
import functools

import jax
import jax.numpy as jnp
from jax import lax
from jax.experimental import pallas as pl
from jax.experimental.pallas import tpu as pltpu
from jax.experimental.pallas import tpu_sc as plsc

N_NODES = 10000
N_EDGES = 320000
D = 128
DH = D // 2
EPS = 1e-5

NC = 2
NS = 16
EPT = N_EDGES // NS
CHUNK = 125
NCHUNK = EPT // CHUNK
RING = 5
ZCH = 80
NODE_CHUNKS = N_NODES // ZCH


def _sc_segment_sum(h2, src4, dst3):
    mesh = plsc.VectorSubcoreMesh(core_axis_name="c", subcore_axis_name="s")

    @functools.partial(
        pl.kernel,
        out_type=jax.ShapeDtypeStruct((NC, N_NODES, DH), jnp.float32),
        mesh=mesh,
        compiler_params=pltpu.CompilerParams(use_tc_tiling_on_sc=False),
        scratch_types=[
            pltpu.VMEM((NCHUNK, CHUNK), jnp.int32),
            pltpu.VMEM((NCHUNK, CHUNK), jnp.int32),
            [pltpu.VMEM((CHUNK, DH), jnp.float32) for _ in range(RING)],
            pltpu.VMEM((ZCH, DH), jnp.float32),
            pltpu.VMEM_SHARED((N_NODES, DH), jnp.float32),
            [pltpu.SemaphoreType.DMA for _ in range(RING)],
            [pltpu.SemaphoreType.DMA for _ in range(RING)],
        ],
    )
    def k(h_hbm, src_hbm, dst_hbm, out_hbm,
          src_v, dst_v, rows, zbuf, acc, gsem, ssem):
        c = lax.axis_index("c")
        s = lax.axis_index("s")

        pltpu.sync_copy(src_hbm.at[c, s], src_v)
        pltpu.sync_copy(dst_hbm.at[s], dst_v)

        def zstore(i, carry):
            zbuf[i // (DH // 16), pl.ds((i % (DH // 16)) * 16, 16)] = (
                jnp.zeros((16,), jnp.float32))
            return carry
        lax.fori_loop(0, ZCH * (DH // 16), zstore, 0)

        def strided_node_chunks(body):
            def it(i, carry):
                cid = s + i * NS

                @pl.when(cid < NODE_CHUNKS)
                def _():
                    body(cid)
                return carry
            lax.fori_loop(0, (NODE_CHUNKS + NS - 1) // NS, it, 0)

        strided_node_chunks(
            lambda cid: pltpu.sync_copy(zbuf, acc.at[pl.ds(cid * ZCH, ZCH)]))
        plsc.subcore_barrier()

        for b in range(RING):
            pltpu.async_copy(h_hbm.at[src_v.at[b]], rows[b], gsem[b])

        def block(jb, carry):
            for b in range(RING):
                j = jb * RING + b
                pltpu.make_async_copy(
                    h_hbm.at[src_v.at[j]], rows[b], gsem[b]).wait()
                pltpu.async_copy(
                    rows[b], acc.at[dst_v.at[j]], ssem[b], add=True)
            for b in range(RING):
                j = jb * RING + b
                pltpu.make_async_copy(
                    rows[b], acc.at[dst_v.at[j]], ssem[b]).wait()
                jn = j + RING

                @pl.when(jn < NCHUNK)
                def _():
                    pltpu.async_copy(
                        h_hbm.at[src_v.at[jn]], rows[b], gsem[b])
            return carry
        lax.fori_loop(0, NCHUNK // RING, block, 0)
        plsc.subcore_barrier()

        strided_node_chunks(
            lambda cid: pltpu.sync_copy(
                acc.at[pl.ds(cid * ZCH, ZCH)],
                out_hbm.at[c, pl.ds(cid * ZCH, ZCH)]))

    return k(h2, src4, dst3)


def _bn_body(parts_ref, gamma_ref, beta_ref, out_ref):
    agg = jnp.concatenate([parts_ref[0], parts_ref[1]], axis=1)
    mean = jnp.mean(agg, axis=0, keepdims=True)
    cent = agg - mean
    var = jnp.mean(cent * cent, axis=0, keepdims=True)
    out_ref[...] = cent * lax.rsqrt(var + EPS) * gamma_ref[...] + beta_ref[...]


def kernel(h, edge_index, gamma, beta):
    h2 = h.reshape(2 * N_NODES, DH)
    src2 = edge_index[0] * 2
    src4 = jnp.stack([src2, src2 + 1]).reshape(NC, NS, NCHUNK, CHUNK)
    dst3 = edge_index[1].reshape(NS, NCHUNK, CHUNK)
    parts = _sc_segment_sum(h2, src4, dst3)
    return pl.pallas_call(
        _bn_body,
        out_shape=jax.ShapeDtypeStruct((N_NODES, D), jnp.float32),
    )(parts, gamma.reshape(1, D), beta.reshape(1, D))


# speedup vs baseline: 13.0044x; 1.1173x over previous
import functools

import jax
import jax.numpy as jnp
from jax import lax
from jax.experimental import pallas as pl
from jax.experimental.pallas import tpu as pltpu
from jax.experimental.pallas import tpu_sc as plsc

N_NODES = 10000
N_EDGES = 320000
D = 128
DH = D // 2
EPS = 1e-5

NC = 2
NS = 16
EPT = N_EDGES // NS
CHUNK = 80
NCHUNK = EPT // CHUNK
RING = 5
ZCH = 80
NODE_CHUNKS = N_NODES // ZCH


def _sc_segment_sum(h2, src, dst):
    mesh = plsc.VectorSubcoreMesh(core_axis_name="c", subcore_axis_name="s")

    @functools.partial(
        pl.kernel,
        out_type=jax.ShapeDtypeStruct((N_NODES, D), jnp.float32),
        mesh=mesh,
        compiler_params=pltpu.CompilerParams(use_tc_tiling_on_sc=False),
        scratch_types=[
            pltpu.VMEM((EPT,), jnp.int32),
            pltpu.VMEM((EPT,), jnp.int32),
            [pltpu.VMEM((CHUNK,), jnp.int32) for _ in range(RING)],
            [pltpu.VMEM((CHUNK,), jnp.int32) for _ in range(RING)],
            [pltpu.VMEM((CHUNK, DH), jnp.float32) for _ in range(RING)],
            pltpu.VMEM((ZCH, DH), jnp.float32),
            pltpu.VMEM_SHARED((N_NODES, DH), jnp.float32),
            [pltpu.SemaphoreType.DMA for _ in range(RING)],
            [pltpu.SemaphoreType.DMA for _ in range(RING)],
        ],
    )
    def k(h_hbm, src_hbm, dst_hbm, out_hbm,
          src_v, dst_v, sidx, didx, rows, zbuf, acc, gsem, ssem):
        c = lax.axis_index("c")
        s = lax.axis_index("s")

        pltpu.sync_copy(src_hbm.at[pl.ds(s * EPT, EPT)], src_v)
        pltpu.sync_copy(dst_hbm.at[pl.ds(s * EPT, EPT)], dst_v)

        def zstore(i, carry):
            zbuf[i // (DH // 16), pl.ds((i % (DH // 16)) * 16, 16)] = (
                jnp.zeros((16,), jnp.float32))
            return carry
        lax.fori_loop(0, ZCH * (DH // 16), zstore, 0)

        def strided_node_chunks(body):
            def it(i, carry):
                cid = s + i * NS

                @pl.when(cid < NODE_CHUNKS)
                def _():
                    body(cid)
                return carry
            lax.fori_loop(0, (NODE_CHUNKS + NS - 1) // NS, it, 0)

        strided_node_chunks(
            lambda cid: pltpu.sync_copy(zbuf, acc.at[pl.ds(cid * ZCH, ZCH)]))
        plsc.subcore_barrier()

        def prep_idx(b, j):
            for q in range(CHUNK // 16):
                e0 = j * CHUNK + q * 16
                sidx[b][pl.ds(q * 16, 16)] = src_v[pl.ds(e0, 16)] * 2 + c
                didx[b][pl.ds(q * 16, 16)] = dst_v[pl.ds(e0, 16)]

        for b in range(RING):
            prep_idx(b, b)
            pltpu.async_copy(h_hbm.at[sidx[b]], rows[b], gsem[b])

        def block(jb, carry):
            for b in range(RING):
                pltpu.make_async_copy(
                    h_hbm.at[sidx[b]], rows[b], gsem[b]).wait()
                pltpu.async_copy(
                    rows[b], acc.at[didx[b]], ssem[b], add=True)
            for b in range(RING):
                j = jb * RING + b
                pltpu.make_async_copy(
                    rows[b], acc.at[didx[b]], ssem[b]).wait()
                jn = j + RING

                @pl.when(jn < NCHUNK)
                def _():
                    prep_idx(b, jn)
                    pltpu.async_copy(h_hbm.at[sidx[b]], rows[b], gsem[b])
            return carry
        lax.fori_loop(0, NCHUNK // RING, block, 0)
        plsc.subcore_barrier()

        strided_node_chunks(
            lambda cid: pltpu.sync_copy(
                acc.at[pl.ds(cid * ZCH, ZCH)],
                out_hbm.at[pl.ds(cid * ZCH, ZCH), pl.ds(c * DH, DH)]))

    return k(h2, src, dst)


def _bn_body(agg_ref, gamma_ref, beta_ref, out_ref):
    agg = agg_ref[...]
    mean = jnp.mean(agg, axis=0, keepdims=True)
    cent = agg - mean
    var = jnp.mean(cent * cent, axis=0, keepdims=True)
    out_ref[...] = cent * lax.rsqrt(var + EPS) * gamma_ref[...] + beta_ref[...]


def kernel(h, edge_index, gamma, beta):
    h2 = h.reshape(2 * N_NODES, DH)
    agg = _sc_segment_sum(h2, edge_index[0], edge_index[1])
    return pl.pallas_call(
        _bn_body,
        out_shape=jax.ShapeDtypeStruct((N_NODES, D), jnp.float32),
    )(agg, gamma.reshape(1, D), beta.reshape(1, D))
